# baseline (speedup 1.0000x reference)
import jax
import jax.numpy as jnp
from jax import lax
from jax.experimental import pallas as pl
from jax.experimental.pallas import tpu as pltpu
from jax.experimental.pallas import tpu_sc as plsc

_N = 65536
_C = 1000
_NB = 20
_NW = 32
_RW = _N // _NW
_L = 16
_CH = 16
_NCH = _RW // _CH
_NBUF = 4
_NK = _C // _L


def _scf_body(x_hbm, lab_hbm, bnd_hbm, part_hbm,
              buf0, buf1, buf2, buf3, lab_v, bnd_v, accum,
              sem0, sem1, sem2, sem3):
    c = lax.axis_index("c")
    s = lax.axis_index("s")
    w = s * 2 + c
    rbase = w * _RW

    bufs = [buf0, buf1, buf2, buf3]
    sems = [sem0, sem1, sem2, sem3]

    pltpu.sync_copy(lab_hbm.at[pl.ds(rbase, _RW)], lab_v)
    pltpu.sync_copy(bnd_hbm, bnd_v)

    zeros = jnp.zeros((_L,), jnp.float32)
    ones = jnp.ones((_L,), jnp.float32)
    for b in range(_NB):
        for q in range(3):
            accum[pl.ds((b * 3 + q) * _L, _L)] = zeros

    bv0 = bnd_v[pl.ds(0, _L)]
    bv1 = bnd_v[pl.ds(_L, _L)]
    bs = [bv0[j] for j in range(_L)] + [bv1[j] for j in range(_NB + 1 - _L)]

    def start(buf, sem, ch):
        pltpu.make_async_copy(
            x_hbm.at[pl.ds(rbase + ch * _CH, _CH), :], buf, sem
        ).start()

    def drain(buf, sem):
        pltpu.make_async_copy(
            x_hbm.at[pl.ds(rbase, _CH), :], buf, sem
        ).wait()

    for b in range(_NBUF):
        start(bufs[b], sems[b], b)

    rows = lax.iota(jnp.int32, _L)
    neginf = jnp.full((_L,), -jnp.inf, jnp.float32)
    zi = jnp.zeros((_L,), jnp.int32)
    _UN = 8

    def process(buf, ch):
        def jstep(i, carry):
            curmax, curcol, colv = carry
            for _ in range(_UN):
                v = plsc.load_gather(buf, [rows, colv])
                m = v > curmax
                curmax = jnp.where(m, v, curmax)
                curcol = jnp.where(m, colv, curcol)
                colv = colv + 1
            return curmax, curcol, colv

        curmax, curcol, _ = lax.fori_loop(
            0, _C // _UN, jstep, (neginf, zi, zi))
        confv = curmax
        colv = curcol

        lab16 = lab_v[pl.ds(ch * _CH, _CH)]
        av = (colv == lab16).astype(jnp.float32)
        cv = confv
        for b in range(_NB):
            m = (cv > bs[b]) & (cv <= bs[b + 1])
            plsc.addupdate(accum.at[pl.ds((b * 3 + 0) * _L, _L)],
                           jnp.where(m, ones, zeros))
            plsc.addupdate(accum.at[pl.ds((b * 3 + 1) * _L, _L)],
                           jnp.where(m, cv, zeros))
            plsc.addupdate(accum.at[pl.ds((b * 3 + 2) * _L, _L)],
                           jnp.where(m, av, zeros))

    def iter_body(it, carry):
        for b in range(_NBUF):
            ch = it * _NBUF + b
            drain(bufs[b], sems[b])
            process(bufs[b], ch)
            nxt = ch + _NBUF

            @pl.when(nxt < _NCH)
            def _():
                start(bufs[b], sems[b], nxt)
        return carry

    lax.fori_loop(0, _NCH // _NBUF, iter_body, 0)

    pltpu.sync_copy(accum, part_hbm.at[pl.ds(w * _NB * 3 * _L, _NB * 3 * _L)])


def _scf_stage(outputs, labels, boundaries):
    mesh = plsc.VectorSubcoreMesh(core_axis_name="c", subcore_axis_name="s")
    return pl.kernel(
        _scf_body,
        out_type=jax.ShapeDtypeStruct((_NW * _NB * 3 * _L,), jnp.float32),
        mesh=mesh,
        compiler_params=pltpu.CompilerParams(needs_layout_passes=False),
        scratch_types=[
            pltpu.VMEM((_CH, _C), jnp.float32),
            pltpu.VMEM((_CH, _C), jnp.float32),
            pltpu.VMEM((_CH, _C), jnp.float32),
            pltpu.VMEM((_CH, _C), jnp.float32),
            pltpu.VMEM((_RW,), jnp.int32),
            pltpu.VMEM((32,), jnp.float32),
            pltpu.VMEM((_NB * 3 * _L,), jnp.float32),
            pltpu.SemaphoreType.DMA,
            pltpu.SemaphoreType.DMA,
            pltpu.SemaphoreType.DMA,
            pltpu.SemaphoreType.DMA,
        ],
    )(outputs, labels, boundaries)


@jax.jit
def kernel(outputs, labels):
    boundaries = jnp.linspace(0.0, 1.0, _NB + 1)
    bnd = jnp.concatenate([boundaries, jnp.full((32 - _NB - 1,), 2.0,
                                                jnp.float32)])
    parts = _scf_stage(outputs, labels, bnd).reshape(_NW, _NB, 3, _L)
    sums = jnp.sum(parts, axis=(0, 3))
    cnt = sums[:, 0]
    conf_s = sums[:, 1]
    acc_s = sums[:, 2]
    safe = jnp.maximum(cnt, 1.0)
    acc_in_bin = jnp.where(cnt > 0, acc_s / safe, 0.0)
    conf_in_bin = jnp.where(cnt > 0, conf_s / safe, 0.0)
    ece = jnp.sum(jnp.abs(conf_in_bin - acc_in_bin) * (cnt / _N))
    return ece.reshape(1)
